# baseline (speedup 1.0000x reference)
import functools

import jax
import jax.numpy as jnp
from jax import lax
from jax.experimental import pallas as pl
from jax.experimental.pallas import tpu as pltpu
from jax.experimental.pallas import tpu_sc as plsc

N = 5000
NP = 5120
BR = 200
RB = 640
MAXN = 1000
RADIUS = 0.1

NWORK = 32
CHUNK = NP // NWORK
HALF = CHUNK // 2
OUT_PAD = MAXN + NWORK


def _mask_body(d_ref, srow_ref, scol_ref, v_ref):
    rel = (scol_ref[...] >= srow_ref[...]) | (d_ref[...] >= RADIUS)
    ok = jnp.min(rel.astype(jnp.float32), axis=1, keepdims=True)
    v_ref[...] = scol_ref[...] * ok


def _rank_body(vcol_ref, vrow_ref, rank_ref):
    b = pl.program_id(0)
    vb = vcol_ref[...]
    vr = vrow_ref[...]
    jt = lax.broadcasted_iota(jnp.int32, (RB, NP), 1)
    it = b * RB + lax.broadcasted_iota(jnp.int32, (RB, NP), 0)
    beats = (vr > vb) | ((vr == vb) & (jt < it))
    rank = jnp.sum(beats.astype(jnp.float32), axis=1, keepdims=True)
    rank_ref[...] = rank.astype(jnp.int32)


def _sc_scatter_body(rank_hbm, out_hbm, idx_a, idx_b, val_a, val_b, sem):
    c = lax.axis_index("c")
    s = lax.axis_index("s")
    wid = s * 2 + c
    base = wid * CHUNK
    pltpu.sync_copy(rank_hbm.at[pl.ds(base, HALF)], idx_a)
    pltpu.sync_copy(rank_hbm.at[pl.ds(base + HALF, HALF)], idx_b)
    junk = MAXN + wid
    for ref in (idx_a, idx_b):
        for k in range(HALF // 16):
            r = ref[pl.ds(k * 16, 16)]
            ref[pl.ds(k * 16, 16)] = jnp.where(r < MAXN, r, junk)
    for k in range(HALF // 16):
        lanes = lax.iota(jnp.int32, 16)
        val_a[pl.ds(k * 16, 16)] = base + k * 16 + lanes
        val_b[pl.ds(k * 16, 16)] = base + HALF + k * 16 + lanes
    cp_a = pltpu.async_copy(val_a, out_hbm.at[idx_a], sem)
    cp_b = pltpu.async_copy(val_b, out_hbm.at[idx_b], sem)
    cp_a.wait()
    cp_b.wait()


def kernel(dists, scores, max_num):
    del max_num
    d2 = dists.reshape(N, N)
    srow = scores.reshape(1, N)
    scol = scores.reshape(N, 1)

    v = pl.pallas_call(
        _mask_body,
        grid=(N // BR,),
        in_specs=[
            pl.BlockSpec((BR, N), lambda i: (i, 0)),
            pl.BlockSpec((1, N), lambda i: (0, 0)),
            pl.BlockSpec((BR, 1), lambda i: (i, 0)),
        ],
        out_specs=pl.BlockSpec((BR, 1), lambda i: (i, 0)),
        out_shape=jax.ShapeDtypeStruct((N, 1), jnp.float32),
    )(d2, srow, scol)

    vp = jnp.concatenate(
        [v, jnp.full((NP - N, 1), -1.0, jnp.float32)], axis=0)
    vrow = vp.reshape(1, NP)

    rank = pl.pallas_call(
        _rank_body,
        grid=(NP // RB,),
        in_specs=[
            pl.BlockSpec((RB, 1), lambda b: (b, 0)),
            pl.BlockSpec((1, NP), lambda b: (0, 0)),
        ],
        out_specs=pl.BlockSpec((RB, 1), lambda b: (b, 0)),
        out_shape=jax.ShapeDtypeStruct((NP, 1), jnp.int32),
    )(vp, vrow)

    scatter = functools.partial(
        pl.kernel,
        mesh=plsc.VectorSubcoreMesh(core_axis_name="c", subcore_axis_name="s"),
        out_type=jax.ShapeDtypeStruct((OUT_PAD,), jnp.int32),
        scratch_types=[
            pltpu.VMEM((HALF,), jnp.int32),
            pltpu.VMEM((HALF,), jnp.int32),
            pltpu.VMEM((HALF,), jnp.int32),
            pltpu.VMEM((HALF,), jnp.int32),
            pltpu.SemaphoreType.DMA,
        ],
    )(_sc_scatter_body)

    picked = scatter(rank.reshape(NP))
    return picked[:MAXN].reshape(1, MAXN)


# speedup vs baseline: 6.8297x; 6.8297x over previous
import functools

import jax
import jax.numpy as jnp
from jax import lax
from jax.experimental import pallas as pl
from jax.experimental.pallas import tpu as pltpu

N = 5000
NP = 5120
BR = 200
RB = 640
KP = 1024
MAXN = 1000
RADIUS = 0.1


def _mask_body(d_ref, srow_ref, scol_ref, v_ref):
    rel = (scol_ref[...] >= srow_ref[...]) | (d_ref[...] >= RADIUS)
    ok = jnp.min(rel.astype(jnp.float32), axis=1, keepdims=True)
    v_ref[...] = scol_ref[...] * ok


def _rank_body(vcol_ref, vrow_ref, out_ref):
    b = pl.program_id(0)
    vb = vcol_ref[...]
    vr = vrow_ref[...]
    jt = lax.broadcasted_iota(jnp.int32, (RB, NP), 1)
    it = b * RB + lax.broadcasted_iota(jnp.int32, (RB, NP), 0)
    beats = (vr > vb) | ((vr == vb) & (jt < it))
    rank = jnp.sum(beats.astype(jnp.float32), axis=1, keepdims=True)
    rank = rank.astype(jnp.int32)
    kio = lax.broadcasted_iota(jnp.int32, (RB, KP), 1)
    ival = b * RB + lax.broadcasted_iota(jnp.int32, (RB, KP), 0)
    hit = rank == kio
    contrib = jnp.sum(jnp.where(hit, ival, 0), axis=0, keepdims=True)

    @pl.when(b == 0)
    def _():
        out_ref[...] = jnp.zeros((1, KP), jnp.int32)

    out_ref[...] += contrib


def kernel(dists, scores, max_num):
    del max_num
    d2 = dists.reshape(N, N)
    srow = scores.reshape(1, N)
    scol = scores.reshape(N, 1)

    v = pl.pallas_call(
        _mask_body,
        grid=(N // BR,),
        in_specs=[
            pl.BlockSpec((BR, N), lambda i: (i, 0)),
            pl.BlockSpec((1, N), lambda i: (0, 0)),
            pl.BlockSpec((BR, 1), lambda i: (i, 0)),
        ],
        out_specs=pl.BlockSpec((BR, 1), lambda i: (i, 0)),
        out_shape=jax.ShapeDtypeStruct((N, 1), jnp.float32),
    )(d2, srow, scol)

    vp = jnp.concatenate(
        [v, jnp.full((NP - N, 1), -1.0, jnp.float32)], axis=0)
    vrow = vp.reshape(1, NP)

    picked = pl.pallas_call(
        _rank_body,
        grid=(NP // RB,),
        in_specs=[
            pl.BlockSpec((RB, 1), lambda b: (b, 0)),
            pl.BlockSpec((1, NP), lambda b: (0, 0)),
        ],
        out_specs=pl.BlockSpec((1, KP), lambda b: (0, 0)),
        out_shape=jax.ShapeDtypeStruct((1, KP), jnp.int32),
    )(vp, vrow)

    return picked[:, :MAXN]


# speedup vs baseline: 7.0623x; 1.0341x over previous
import jax
import jax.numpy as jnp
from jax import lax
from jax.experimental import pallas as pl
from jax.experimental.pallas import tpu as pltpu

N = 5000
BR = 128
NB = 40
NP = NB * BR
KP = 1024
MAXN = 1000
RADIUS = 0.1
PREC = lax.Precision.HIGHEST


def _fused_body(d_ref, srow_ref, sblk_ref, out_ref, vrow_s, accr_s, accc_s):
    i = pl.program_id(0)

    @pl.when(i == 0)
    def _init():
        accc_s[...] = jnp.zeros((1, NP), jnp.float32)

    @pl.when(i < NB)
    def _scan():
        srow = srow_ref[...]
        sblk = sblk_ref[...]
        scol = jnp.transpose(sblk, (1, 0))
        t = jnp.where(d_ref[...] < RADIUS, srow, -1.0)
        m = jnp.max(t, axis=1, keepdims=True)
        gr = i * BR + lax.broadcasted_iota(jnp.int32, (BR, 1), 0)
        keep = (scol >= m) & (gr < N)
        vb = jnp.where(keep, scol, jnp.where(gr < N, 0.0, -1.0))
        vbr = jnp.transpose(vb, (1, 0))
        vrow_s[:, pl.ds(i * BR, BR)] = vbr

        jt = lax.broadcasted_iota(jnp.int32, (BR, BR), 1)
        it = lax.broadcasted_iota(jnp.int32, (BR, BR), 0)
        diag = ((vbr > vb) | ((vbr == vb) & (jt < it))).astype(jnp.float32)

        def body(k, gs):
            va = vrow_s[:, pl.ds(k * BR, BR)]
            g = (va >= vb).astype(jnp.float32)
            cs = jnp.sum(g, axis=0, keepdims=True)
            accc_s[:, pl.ds(k * BR, BR)] += float(BR) - cs
            return gs + g

        gsum = lax.fori_loop(0, i, body, diag)
        accr_s[pl.ds(i * BR, BR), :] = jnp.sum(gsum, axis=1, keepdims=True)

    @pl.when(i == NB)
    def _select():
        acct = lax.dot_general(accc_s[...], jnp.ones((1, 1), jnp.float32),
                               (((0,), (0,)), ((), ())),
                               precision=PREC)
        rank = (accr_s[...] + acct).astype(jnp.int32)
        acc = jnp.zeros((1, KP), jnp.float32)
        sel_r = 640
        for t in range(NP // sel_r):
            blk = rank[t * sel_r:(t + 1) * sel_r]
            kio = lax.broadcasted_iota(jnp.int32, (sel_r, KP), 1)
            hit = (blk == kio).astype(jnp.float32)
            ival = (t * sel_r
                    + lax.broadcasted_iota(jnp.int32, (1, sel_r), 1)
                    ).astype(jnp.float32)
            acc += lax.dot_general(
                ival, hit, (((1,), (0,)), ((), ())), precision=PREC)
        out_ref[...] = acc.astype(jnp.int32)


def kernel(dists, scores, max_num):
    del max_num
    d2 = dists.reshape(N, N)
    srow = scores.reshape(1, N)

    picked = pl.pallas_call(
        _fused_body,
        grid=(NB + 1,),
        in_specs=[
            pl.BlockSpec((BR, N), lambda i: (jnp.minimum(i, NB - 1), 0)),
            pl.BlockSpec((1, N), lambda i: (0, 0)),
            pl.BlockSpec((1, BR), lambda i: (0, jnp.minimum(i, NB - 1))),
        ],
        out_specs=pl.BlockSpec((1, KP), lambda i: (0, 0)),
        out_shape=jax.ShapeDtypeStruct((1, KP), jnp.int32),
        scratch_shapes=[
            pltpu.VMEM((1, NP), jnp.float32),
            pltpu.VMEM((NP, 1), jnp.float32),
            pltpu.VMEM((1, NP), jnp.float32),
        ],
    )(d2, srow, srow)

    return picked[:, :MAXN]


# speedup vs baseline: 9.5442x; 1.3514x over previous
import jax
import jax.numpy as jnp
from jax import lax
from jax.experimental import pallas as pl
from jax.experimental.pallas import tpu as pltpu

N = 5000
BR = 128
NB = 40
NP = NB * BR
HC = 2560
KP = 1024
MAXN = 1000
RADIUS = 0.1


def _fused_body(dl_ref, dr_ref, srow_ref, sblk_ref, out_ref,
                vrow_s, accr_s, accc_s, spad_s):
    i = pl.program_id(0)

    @pl.when(i == 0)
    def _init():
        accc_s[...] = jnp.zeros((1, NP), jnp.float32)
        spad_s[...] = jnp.concatenate(
            [srow_ref[...], jnp.full((1, NP - N), -1.0, jnp.float32)],
            axis=1)

    @pl.when(i < NB)
    def _scan():
        sblk = sblk_ref[...]
        scol = jnp.transpose(sblk, (1, 0))
        tl = jnp.where(dl_ref[...] < RADIUS, spad_s[:, :HC], -1.0)
        tr = jnp.where(dr_ref[...] < RADIUS, spad_s[:, HC:], -1.0)
        m = jnp.maximum(jnp.max(tl, axis=1, keepdims=True),
                        jnp.max(tr, axis=1, keepdims=True))
        gr = i * BR + lax.broadcasted_iota(jnp.int32, (BR, 1), 0)
        keep = (scol >= m) & (gr < N)
        vb = jnp.where(keep, scol, jnp.where(gr < N, 0.0, -1.0))
        vbr = jnp.transpose(vb, (1, 0))
        vrow_s[:, pl.ds(i * BR, BR)] = vbr

        jt = lax.broadcasted_iota(jnp.int32, (BR, BR), 1)
        it = lax.broadcasted_iota(jnp.int32, (BR, BR), 0)
        diag = ((vbr > vb) | ((vbr == vb) & (jt < it))).astype(jnp.float32)

        def body4(c, gs):
            va = vrow_s[:, pl.ds(c * 4 * BR, 4 * BR)]
            g = (va >= vb).astype(jnp.float32)
            cs = jnp.sum(g, axis=0, keepdims=True)
            accc_s[:, pl.ds(c * 4 * BR, 4 * BR)] += float(BR) - cs
            return (gs + g[:, :BR] + g[:, BR:2 * BR]
                    + g[:, 2 * BR:3 * BR] + g[:, 3 * BR:])

        def body1(k, gs):
            va = vrow_s[:, pl.ds(k * BR, BR)]
            g = (va >= vb).astype(jnp.float32)
            cs = jnp.sum(g, axis=0, keepdims=True)
            accc_s[:, pl.ds(k * BR, BR)] += float(BR) - cs
            return gs + g

        nc4 = i // 4
        gs = lax.fori_loop(0, nc4, body4, diag)
        gsum = lax.fori_loop(nc4 * 4, i, body1, gs)
        accr_s[pl.ds(i * BR, BR), :] = jnp.sum(gsum, axis=1, keepdims=True)

    @pl.when(i == NB)
    def _select():
        acct = jnp.transpose(accc_s[...], (1, 0))
        rank = (accr_s[...] + acct).astype(jnp.int32)
        sel_r = 640
        kio = lax.broadcasted_iota(jnp.int32, (sel_r, KP), 1)
        isub = lax.broadcasted_iota(jnp.int32, (sel_r, KP), 0)
        acc = jnp.zeros((1, KP), jnp.int32)
        for t in range(NP // sel_r):
            blk = rank[t * sel_r:(t + 1) * sel_r]
            hitv = jnp.where(blk == kio, isub + t * sel_r, 0)
            acc = acc + jnp.sum(hitv, axis=0, keepdims=True)
        out_ref[...] = acc


def kernel(dists, scores, max_num):
    del max_num
    d2 = dists.reshape(N, N)
    srow = scores.reshape(1, N)

    picked = pl.pallas_call(
        _fused_body,
        grid=(NB + 1,),
        in_specs=[
            pl.BlockSpec((BR, HC), lambda i: (jnp.minimum(i, NB - 1), 0)),
            pl.BlockSpec((BR, HC), lambda i: (jnp.minimum(i, NB - 1), 1)),
            pl.BlockSpec((1, N), lambda i: (0, 0)),
            pl.BlockSpec((1, BR), lambda i: (0, jnp.minimum(i, NB - 1))),
        ],
        out_specs=pl.BlockSpec((1, KP), lambda i: (0, 0)),
        out_shape=jax.ShapeDtypeStruct((1, KP), jnp.int32),
        scratch_shapes=[
            pltpu.VMEM((1, NP), jnp.float32),
            pltpu.VMEM((NP, 1), jnp.float32),
            pltpu.VMEM((1, NP), jnp.float32),
            pltpu.VMEM((1, NP), jnp.float32),
        ],
    )(d2, d2, srow, srow)

    return picked[:, :MAXN]


# speedup vs baseline: 9.8572x; 1.0328x over previous
import jax
import jax.numpy as jnp
from jax import lax
from jax.experimental import pallas as pl
from jax.experimental.pallas import tpu as pltpu

N = 5000
BR = 128
NB = 40
NP = NB * BR
QC = 1280
NQ = 4
KP = 1024
MAXN = 1000
RADIUS = 0.1


def _fused_body(d0_ref, d1_ref, d2_ref, d3_ref, srow_ref, sblk_ref, out_ref,
                vrow_s, accr_s, accc_s, spad_s):
    i = pl.program_id(0)

    @pl.when(i == 0)
    def _init():
        accc_s[...] = jnp.zeros((1, NP), jnp.float32)
        spad_s[...] = jnp.concatenate(
            [srow_ref[...], jnp.full((1, NP - N), -1.0, jnp.float32)],
            axis=1)

    @pl.when(i < NB)
    def _scan():
        sblk = sblk_ref[...]
        scol = jnp.transpose(sblk, (1, 0))
        m = jnp.full((BR, 1), -1.0, jnp.float32)
        for q, dq in enumerate((d0_ref, d1_ref, d2_ref, d3_ref)):
            tq = jnp.where(dq[...] < RADIUS,
                           spad_s[:, q * QC:(q + 1) * QC], -1.0)
            m = jnp.maximum(m, jnp.max(tq, axis=1, keepdims=True))
        gr = i * BR + lax.broadcasted_iota(jnp.int32, (BR, 1), 0)
        keep = (scol >= m) & (gr < N)
        vb = jnp.where(keep, scol, jnp.where(gr < N, 0.0, -1.0))
        vbr = jnp.transpose(vb, (1, 0))
        vrow_s[:, pl.ds(i * BR, BR)] = vbr

        jt = lax.broadcasted_iota(jnp.int32, (BR, BR), 1)
        it = lax.broadcasted_iota(jnp.int32, (BR, BR), 0)
        diag = ((vbr > vb) | ((vbr == vb) & (jt < it))).astype(jnp.float32)

        def body4(c, gs):
            va = vrow_s[:, pl.ds(c * 4 * BR, 4 * BR)]
            g = (va >= vb).astype(jnp.float32)
            cs = jnp.sum(g, axis=0, keepdims=True)
            accc_s[:, pl.ds(c * 4 * BR, 4 * BR)] += float(BR) - cs
            return (gs + g[:, :BR] + g[:, BR:2 * BR]
                    + g[:, 2 * BR:3 * BR] + g[:, 3 * BR:])

        def body1(k, gs):
            va = vrow_s[:, pl.ds(k * BR, BR)]
            g = (va >= vb).astype(jnp.float32)
            cs = jnp.sum(g, axis=0, keepdims=True)
            accc_s[:, pl.ds(k * BR, BR)] += float(BR) - cs
            return gs + g

        nc4 = i // 4
        gs = lax.fori_loop(0, nc4, body4, diag)
        gsum = lax.fori_loop(nc4 * 4, i, body1, gs)
        accr_s[pl.ds(i * BR, BR), :] = jnp.sum(gsum, axis=1, keepdims=True)

    @pl.when(i == NB)
    def _select():
        acct = jnp.transpose(accc_s[...], (1, 0))
        rank = (accr_s[...] + acct).astype(jnp.int32)
        sel_r = 640
        kio = lax.broadcasted_iota(jnp.int32, (sel_r, KP), 1)
        isub = lax.broadcasted_iota(jnp.int32, (sel_r, KP), 0)
        acc = jnp.zeros((1, KP), jnp.int32)
        for t in range(NP // sel_r):
            blk = rank[t * sel_r:(t + 1) * sel_r]
            hitv = jnp.where(blk == kio, isub + t * sel_r, 0)
            acc = acc + jnp.sum(hitv, axis=0, keepdims=True)
        out_ref[...] = acc


def kernel(dists, scores, max_num):
    del max_num
    d2 = dists.reshape(N, N)
    srow = scores.reshape(1, N)

    picked = pl.pallas_call(
        _fused_body,
        grid=(NB + 1,),
        in_specs=[
            pl.BlockSpec((BR, QC),
                         lambda i, q=q: (jnp.minimum(i, NB - 1), q))
            for q in range(NQ)
        ] + [
            pl.BlockSpec((1, N), lambda i: (0, 0)),
            pl.BlockSpec((1, BR), lambda i: (0, jnp.minimum(i, NB - 1))),
        ],
        out_specs=pl.BlockSpec((1, KP), lambda i: (0, 0)),
        out_shape=jax.ShapeDtypeStruct((1, KP), jnp.int32),
        scratch_shapes=[
            pltpu.VMEM((1, NP), jnp.float32),
            pltpu.VMEM((NP, 1), jnp.float32),
            pltpu.VMEM((1, NP), jnp.float32),
            pltpu.VMEM((1, NP), jnp.float32),
        ],
    )(d2, d2, d2, d2, srow, srow)

    return picked[:, :MAXN]
